# Initial kernel scaffold; baseline (speedup 1.0000x reference)
import jax
import jax.numpy as jnp
from jax.experimental import pallas as pl


def kernel(z, z_embed_weight):
    raise NotImplementedError("write your pallas kernel here")


# speedup vs baseline: 1.1618x; 1.1618x over previous
import functools

import jax
import jax.numpy as jnp
from jax import lax
from jax.experimental import pallas as pl
from jax.experimental.pallas import tpu as pltpu
from jax.experimental.pallas import tpu_sc as plsc

N_NODE = 100000
EMBED_DIM = 128
CHUNK = 128
NUM_WORKERS = 32
FULL_CHUNKS = N_NODE // CHUNK
TAIL = N_NODE - FULL_CHUNKS * CHUNK
CHUNKS_PER_WORKER = (FULL_CHUNKS + NUM_WORKERS - 1) // NUM_WORKERS

_mesh = plsc.VectorSubcoreMesh(core_axis_name="c", subcore_axis_name="s")


@functools.partial(
    pl.kernel,
    mesh=_mesh,
    out_type=jax.ShapeDtypeStruct((N_NODE, EMBED_DIM), jnp.float32),
    scratch_types=[
        pltpu.VMEM((CHUNK,), jnp.int32),
        pltpu.VMEM((CHUNK, EMBED_DIM), jnp.float32),
        pltpu.VMEM((TAIL,), jnp.int32),
        pltpu.VMEM((TAIL, EMBED_DIM), jnp.float32),
        pltpu.SemaphoreType.DMA,
    ],
)
def _embed_sc(z_hbm, w_hbm, out_hbm, idx_v, rows_v, idx_t, rows_t, sem):
    wid = lax.axis_index("s") * 2 + lax.axis_index("c")

    def chunk_body(i, carry):
        cid = wid + i * NUM_WORKERS

        @pl.when(cid < FULL_CHUNKS)
        def _():
            base = cid * CHUNK
            pltpu.sync_copy(z_hbm.at[pl.ds(base, CHUNK)], idx_v)
            pltpu.async_copy(w_hbm.at[idx_v], rows_v, sem).wait()
            pltpu.sync_copy(rows_v, out_hbm.at[pl.ds(base, CHUNK)])

        return carry

    lax.fori_loop(0, CHUNKS_PER_WORKER, chunk_body, 0)

    @pl.when(wid == NUM_WORKERS - 1)
    def _():
        base = FULL_CHUNKS * CHUNK
        pltpu.sync_copy(z_hbm.at[pl.ds(base, TAIL)], idx_t)
        pltpu.async_copy(w_hbm.at[idx_t], rows_t, sem).wait()
        pltpu.sync_copy(rows_t, out_hbm.at[pl.ds(base, TAIL)])


def kernel(z, z_embed_weight):
    return _embed_sc(z.astype(jnp.int32), z_embed_weight)


# speedup vs baseline: 1.1911x; 1.0252x over previous
import functools

import jax
import jax.numpy as jnp
from jax import lax
from jax.experimental import pallas as pl
from jax.experimental.pallas import tpu as pltpu
from jax.experimental.pallas import tpu_sc as plsc

N_NODE = 100000
EMBED_DIM = 128
CHUNK = 128
NUM_WORKERS = 32
FULL_CHUNKS = N_NODE // CHUNK
TAIL = N_NODE - FULL_CHUNKS * CHUNK
TAIL_BASE = FULL_CHUNKS * CHUNK
MAX_SLOTS = 25
NBUF = 4

_mesh = plsc.VectorSubcoreMesh(core_axis_name="c", subcore_axis_name="s")


@functools.partial(
    pl.kernel,
    mesh=_mesh,
    out_type=jax.ShapeDtypeStruct((N_NODE, EMBED_DIM), jnp.float32),
    scratch_types=[
        pltpu.VMEM((MAX_SLOTS * CHUNK,), jnp.int32),
        pltpu.VMEM((CHUNK, EMBED_DIM), jnp.float32),
        pltpu.VMEM((CHUNK, EMBED_DIM), jnp.float32),
        pltpu.VMEM((CHUNK, EMBED_DIM), jnp.float32),
        pltpu.VMEM((CHUNK, EMBED_DIM), jnp.float32),
        pltpu.VMEM((TAIL,), jnp.int32),
        pltpu.VMEM((TAIL, EMBED_DIM), jnp.float32),
        pltpu.SemaphoreType.DMA,
        pltpu.SemaphoreType.DMA,
        pltpu.SemaphoreType.DMA,
        pltpu.SemaphoreType.DMA,
        pltpu.SemaphoreType.DMA,
        pltpu.SemaphoreType.DMA,
        pltpu.SemaphoreType.DMA,
        pltpu.SemaphoreType.DMA,
        pltpu.SemaphoreType.DMA,
        pltpu.SemaphoreType.DMA,
    ],
)
def _embed_sc(z_hbm, w_hbm, out_hbm, idx_all, r0, r1, r2, r3, idx_t, rows_t,
              isem, g0, g1, g2, g3, w0, w1, w2, w3, tsem):
    rows = (r0, r1, r2, r3)
    gsem = (g0, g1, g2, g3)
    wsem = (w0, w1, w2, w3)
    wid = lax.axis_index("s") * 2 + lax.axis_index("c")

    def g_start(s, b):
        return pltpu.async_copy(
            w_hbm.at[idx_all.at[pl.ds(s * CHUNK, CHUNK)]], rows[b], gsem[b])

    def g_wait(b):
        pltpu.make_async_copy(
            out_hbm.at[pl.ds(0, CHUNK)], rows[b], gsem[b]).wait()

    def w_start(s, b):
        return pltpu.async_copy(
            rows[b], out_hbm.at[pl.ds((wid + s * NUM_WORKERS) * CHUNK, CHUNK)],
            wsem[b])

    def w_wait(b):
        pltpu.make_async_copy(
            rows[b], out_hbm.at[pl.ds(0, CHUNK)], wsem[b]).wait()

    preload = [
        pltpu.async_copy(
            z_hbm.at[pl.ds((wid + s * NUM_WORKERS) * CHUNK, CHUNK)],
            idx_all.at[pl.ds(s * CHUNK, CHUNK)], isem)
        for s in range(MAX_SLOTS - 1)
    ]
    for cp in preload:
        cp.wait()

    @pl.when(wid < FULL_CHUNKS - (MAX_SLOTS - 1) * NUM_WORKERS)
    def _():
        s = MAX_SLOTS - 1
        pltpu.sync_copy(
            z_hbm.at[pl.ds((wid + s * NUM_WORKERS) * CHUNK, CHUNK)],
            idx_all.at[pl.ds(s * CHUNK, CHUNK)])

    g_start(0, 0)
    g_start(1, 1)
    g_start(2, 2)

    g_wait(0)
    w_start(0, 0)
    g_start(3, 3)

    g_wait(1)
    w_start(1, 1)
    w_wait(0)
    g_start(4, 0)

    g_wait(2)
    w_start(2, 2)
    w_wait(1)
    g_start(5, 1)

    def loop_body(it, carry):
        for j in range(NBUF):
            b = (3 + j) % NBUF
            s = 3 + it * NBUF + j
            g_wait(b)
            w_start(s, b)

            nb = (2 + j) % NBUF

            @pl.when(wid + (s + 3) * NUM_WORKERS < FULL_CHUNKS)
            def _():
                w_wait(nb)
                g_start(s + 3, nb)

        return carry

    lax.fori_loop(0, 5, loop_body, 0)

    g_wait(3)
    w_start(23, 3)

    @pl.when(wid < FULL_CHUNKS - (MAX_SLOTS - 1) * NUM_WORKERS)
    def _():
        g_wait(0)
        w_start(24, 0)

    @pl.when(wid == NUM_WORKERS - 1)
    def _():
        pltpu.sync_copy(z_hbm.at[pl.ds(TAIL_BASE, TAIL)], idx_t)
        pltpu.async_copy(w_hbm.at[idx_t], rows_t, tsem).wait()
        pltpu.sync_copy(rows_t, out_hbm.at[pl.ds(TAIL_BASE, TAIL)])

    w_wait(0)
    w_wait(1)
    w_wait(2)
    w_wait(3)


def kernel(z, z_embed_weight):
    return _embed_sc(z.astype(jnp.int32), z_embed_weight)


# speedup vs baseline: 5.6609x; 4.7525x over previous
import functools

import jax
import jax.numpy as jnp
from jax import lax
from jax.experimental import pallas as pl
from jax.experimental.pallas import tpu as pltpu
from jax.experimental.pallas import tpu_sc as plsc

N_NODE = 100000
EMBED_DIM = 128
CHUNK = 128
NUM_WORKERS = 32
FULL_CHUNKS = N_NODE // CHUNK
TAIL = N_NODE - FULL_CHUNKS * CHUNK
TAIL_BASE = FULL_CHUNKS * CHUNK
MAX_SLOTS = 25
MAX_Z_ROWS = 37
NBUF = 4

_mesh = plsc.VectorSubcoreMesh(core_axis_name="c", subcore_axis_name="s")


@functools.partial(
    pl.kernel,
    mesh=_mesh,
    out_type=jax.ShapeDtypeStruct((N_NODE, EMBED_DIM), jnp.float32),
    scratch_types=[
        pltpu.VMEM((MAX_SLOTS * CHUNK,), jnp.int32),
        pltpu.VMEM((CHUNK, EMBED_DIM), jnp.float32),
        pltpu.VMEM((CHUNK, EMBED_DIM), jnp.float32),
        pltpu.VMEM((CHUNK, EMBED_DIM), jnp.float32),
        pltpu.VMEM((CHUNK, EMBED_DIM), jnp.float32),
        pltpu.VMEM((TAIL,), jnp.int32),
        pltpu.VMEM((TAIL, EMBED_DIM), jnp.float32),
        pltpu.VMEM_SHARED((MAX_Z_ROWS, EMBED_DIM), jnp.float32),
        pltpu.SemaphoreType.DMA,
        pltpu.SemaphoreType.DMA,
        pltpu.SemaphoreType.DMA,
        pltpu.SemaphoreType.DMA,
        pltpu.SemaphoreType.DMA,
        pltpu.SemaphoreType.DMA,
        pltpu.SemaphoreType.DMA,
        pltpu.SemaphoreType.DMA,
        pltpu.SemaphoreType.DMA,
        pltpu.SemaphoreType.DMA,
    ],
)
def _embed_sc(z_hbm, w_hbm, out_hbm, idx_all, r0, r1, r2, r3, idx_t, rows_t,
              w_sh, isem, g0, g1, g2, g3, w0, w1, w2, w3, tsem):
    rows = (r0, r1, r2, r3)
    gsem = (g0, g1, g2, g3)
    wsem = (w0, w1, w2, w3)
    wid = lax.axis_index("s") * 2 + lax.axis_index("c")

    @pl.when(lax.axis_index("s") == 0)
    def _():
        pltpu.sync_copy(w_hbm, w_sh)

    plsc.subcore_barrier()

    def g_start(s, b):
        return pltpu.async_copy(
            w_sh.at[idx_all.at[pl.ds(s * CHUNK, CHUNK)]], rows[b], gsem[b])

    def g_wait(b):
        pltpu.make_async_copy(
            out_hbm.at[pl.ds(0, CHUNK)], rows[b], gsem[b]).wait()

    def w_start(s, b):
        return pltpu.async_copy(
            rows[b], out_hbm.at[pl.ds((wid + s * NUM_WORKERS) * CHUNK, CHUNK)],
            wsem[b])

    def w_wait(b):
        pltpu.make_async_copy(
            rows[b], out_hbm.at[pl.ds(0, CHUNK)], wsem[b]).wait()

    preload = [
        pltpu.async_copy(
            z_hbm.at[pl.ds((wid + s * NUM_WORKERS) * CHUNK, CHUNK)],
            idx_all.at[pl.ds(s * CHUNK, CHUNK)], isem)
        for s in range(MAX_SLOTS - 1)
    ]
    for cp in preload:
        cp.wait()

    @pl.when(wid < FULL_CHUNKS - (MAX_SLOTS - 1) * NUM_WORKERS)
    def _():
        s = MAX_SLOTS - 1
        pltpu.sync_copy(
            z_hbm.at[pl.ds((wid + s * NUM_WORKERS) * CHUNK, CHUNK)],
            idx_all.at[pl.ds(s * CHUNK, CHUNK)])

    g_start(0, 0)
    g_start(1, 1)
    g_start(2, 2)

    g_wait(0)
    w_start(0, 0)
    g_start(3, 3)

    g_wait(1)
    w_start(1, 1)
    w_wait(0)
    g_start(4, 0)

    g_wait(2)
    w_start(2, 2)
    w_wait(1)
    g_start(5, 1)

    def loop_body(it, carry):
        for j in range(NBUF):
            b = (3 + j) % NBUF
            s = 3 + it * NBUF + j
            g_wait(b)
            w_start(s, b)

            nb = (2 + j) % NBUF

            @pl.when(wid + (s + 3) * NUM_WORKERS < FULL_CHUNKS)
            def _():
                w_wait(nb)
                g_start(s + 3, nb)

        return carry

    lax.fori_loop(0, 5, loop_body, 0)

    g_wait(3)
    w_start(23, 3)

    @pl.when(wid < FULL_CHUNKS - (MAX_SLOTS - 1) * NUM_WORKERS)
    def _():
        g_wait(0)
        w_start(24, 0)

    @pl.when(wid == NUM_WORKERS - 1)
    def _():
        pltpu.sync_copy(z_hbm.at[pl.ds(TAIL_BASE, TAIL)], idx_t)
        pltpu.async_copy(w_sh.at[idx_t], rows_t, tsem).wait()
        pltpu.sync_copy(rows_t, out_hbm.at[pl.ds(TAIL_BASE, TAIL)])

    w_wait(0)
    w_wait(1)
    w_wait(2)
    w_wait(3)


def kernel(z, z_embed_weight):
    return _embed_sc(z.astype(jnp.int32), z_embed_weight)
